# baseline (speedup 1.0000x reference)
import jax
import jax.numpy as jnp
from jax import lax
from jax.experimental import pallas as pl
from jax.experimental.pallas import tpu as pltpu

T = 2048
H = 1024
E = 16
I = 256
TOPK = 2
NG = 4
TG = 2
RSF = 2.5

_NEG = float(jnp.finfo(jnp.float32).min)


def _silu(x):
    return x * jax.nn.sigmoid(x)


def _dotT(a, b):
    return lax.dot_general(a, b, (((1,), (1,)), ((), ())),
                           preferred_element_type=jnp.float32)


def _routing(x, Wg, bias2):
    logits = _dotT(x, Wg)
    scores = jax.nn.sigmoid(logits)
    sfc = scores + bias2
    lane = lax.broadcasted_iota(jnp.int32, (T, E), 1)
    gid = lane // (E // NG)
    gsc = jnp.zeros((T, E), jnp.float32)
    for g in range(NG):
        gs = jnp.where(gid == g, sfc, _NEG)
        m1 = jnp.max(gs, axis=1, keepdims=True)
        m2 = jnp.max(jnp.where(gs >= m1, _NEG, gs), axis=1, keepdims=True)
        gsc = jnp.where(gid == g, m1 + m2, gsc)
    gmax1 = jnp.max(gsc, axis=1, keepdims=True)
    mask1 = gsc >= gmax1
    gmax2 = jnp.max(jnp.where(mask1, _NEG, gsc), axis=1, keepdims=True)
    smask = mask1 | (gsc >= gmax2)
    masked = jnp.where(smask, sfc, _NEG)
    t1 = jnp.max(masked, axis=1, keepdims=True)
    em1 = masked >= t1
    t2 = jnp.max(jnp.where(em1, _NEG, masked), axis=1, keepdims=True)
    sel = em1 | (masked >= t2)
    w = jnp.where(sel, scores, 0.0)
    denom = jnp.sum(w, axis=1, keepdims=True) + 1e-20
    return w / denom * RSF


def _moe_body(x_ref, Wg_ref, b_ref, W1_ref, W3_ref, W2_ref,
              Ws1_ref, Ws3_ref, Ws2_ref, out_ref, comb_ref):
    e = pl.program_id(0)
    x = x_ref[...]
    xb = x.astype(jnp.bfloat16)

    @pl.when(e == 0)
    def _():
        h = (_silu(_dotT(xb, Ws1_ref[...].astype(jnp.bfloat16)))
             * _dotT(xb, Ws3_ref[...].astype(jnp.bfloat16)))
        out_ref[...] = _dotT(h.astype(jnp.bfloat16),
                             Ws2_ref[...].astype(jnp.bfloat16))
        comb_ref[...] = _routing(x, Wg_ref[...], b_ref[...])

    h = (_silu(_dotT(xb, W1_ref[0].astype(jnp.bfloat16)))
         * _dotT(xb, W3_ref[0].astype(jnp.bfloat16)))
    eo = _dotT(h.astype(jnp.bfloat16), W2_ref[0].astype(jnp.bfloat16))
    lane = lax.broadcasted_iota(jnp.int32, (T, E), 1)
    col = jnp.sum(jnp.where(lane == e, comb_ref[...], 0.0),
                  axis=1, keepdims=True)
    out_ref[...] += col * eo


def kernel(hidden_states, Wg, bias, W1, W3, W2, Ws1, Ws3, Ws2):
    bias2 = bias.reshape(1, E)
    out, _ = pl.pallas_call(
        _moe_body,
        grid=(E,),
        in_specs=[
            pl.BlockSpec((T, H), lambda e: (0, 0)),
            pl.BlockSpec((E, H), lambda e: (0, 0)),
            pl.BlockSpec((1, E), lambda e: (0, 0)),
            pl.BlockSpec((1, I, H), lambda e: (e, 0, 0)),
            pl.BlockSpec((1, I, H), lambda e: (e, 0, 0)),
            pl.BlockSpec((1, H, I), lambda e: (e, 0, 0)),
            pl.BlockSpec((I, H), lambda e: (0, 0)),
            pl.BlockSpec((I, H), lambda e: (0, 0)),
            pl.BlockSpec((H, I), lambda e: (0, 0)),
        ],
        out_specs=[
            pl.BlockSpec((T, H), lambda e: (0, 0)),
            pl.BlockSpec((T, E), lambda e: (0, 0)),
        ],
        out_shape=[
            jax.ShapeDtypeStruct((T, H), jnp.float32),
            jax.ShapeDtypeStruct((T, E), jnp.float32),
        ],
        compiler_params=pltpu.CompilerParams(
            dimension_semantics=("arbitrary",)),
    )(hidden_states, Wg, bias2, W1, W3, W2, Ws1, Ws3, Ws2)
    return out


# speedup vs baseline: 1.0603x; 1.0603x over previous
import jax
import jax.numpy as jnp
from jax import lax
from jax.experimental import pallas as pl
from jax.experimental.pallas import tpu as pltpu

T = 2048
H = 1024
E = 16
I = 256
TOPK = 2
NG = 4
TG = 2
RSF = 2.5

EC = 2
NC = E // EC
CI = EC * I

_NEG = float(jnp.finfo(jnp.float32).min)


def _silu(x):
    return x * jax.nn.sigmoid(x)


def _dotT(a, b):
    return lax.dot_general(a, b, (((1,), (1,)), ((), ())),
                           preferred_element_type=jnp.float32)


def _routing(x, Wg, bias2):
    logits = _dotT(x, Wg)
    scores = jax.nn.sigmoid(logits)
    sfc = scores + bias2
    lane = lax.broadcasted_iota(jnp.int32, (T, E), 1)
    gid = lane // (E // NG)
    gsc = jnp.zeros((T, E), jnp.float32)
    for g in range(NG):
        gs = jnp.where(gid == g, sfc, _NEG)
        m1 = jnp.max(gs, axis=1, keepdims=True)
        m2 = jnp.max(jnp.where(gs >= m1, _NEG, gs), axis=1, keepdims=True)
        gsc = jnp.where(gid == g, m1 + m2, gsc)
    gmax1 = jnp.max(gsc, axis=1, keepdims=True)
    mask1 = gsc >= gmax1
    gmax2 = jnp.max(jnp.where(mask1, _NEG, gsc), axis=1, keepdims=True)
    smask = mask1 | (gsc >= gmax2)
    masked = jnp.where(smask, sfc, _NEG)
    t1 = jnp.max(masked, axis=1, keepdims=True)
    em1 = masked >= t1
    t2 = jnp.max(jnp.where(em1, _NEG, masked), axis=1, keepdims=True)
    sel = em1 | (masked >= t2)
    w = jnp.where(sel, scores, 0.0)
    denom = jnp.sum(w, axis=1, keepdims=True) + 1e-20
    return w / denom * RSF


def _moe_body(x_ref, Wg_ref, b_ref, W1_ref, W3_ref, W2_ref,
              Ws1_ref, Ws3_ref, Ws2_ref, out_ref, comb_ref, hc_ref):
    c = pl.program_id(0)
    x = x_ref[...]

    @pl.when(c == 0)
    def _():
        h = _silu(_dotT(x, Ws1_ref[...])) * _dotT(x, Ws3_ref[...])
        out_ref[...] = _dotT(h, Ws2_ref[...])
        comb_ref[...] = _routing(x, Wg_ref[...], b_ref[...])

    h1 = _dotT(x, W1_ref[...])
    h3 = _dotT(x, W3_ref[...])
    comb = comb_ref[...]
    lane = lax.broadcasted_iota(jnp.int32, (T, E), 1)
    for k in range(EC):
        sl = slice(k * I, (k + 1) * I)
        col = jnp.sum(jnp.where(lane == EC * c + k, comb, 0.0),
                      axis=1, keepdims=True)
        hc_ref[:, sl] = _silu(h1[:, sl]) * h3[:, sl] * col
    w2t = jnp.transpose(W2_ref[...], (0, 2, 1)).reshape(CI, H)
    out_ref[...] += jnp.dot(hc_ref[...], w2t,
                            preferred_element_type=jnp.float32)


def kernel(hidden_states, Wg, bias, W1, W3, W2, Ws1, Ws3, Ws2):
    bias2 = bias.reshape(1, E)
    W1r = W1.reshape(E * I, H)
    W3r = W3.reshape(E * I, H)
    out, _ = pl.pallas_call(
        _moe_body,
        grid=(NC,),
        in_specs=[
            pl.BlockSpec((T, H), lambda c: (0, 0)),
            pl.BlockSpec((E, H), lambda c: (0, 0)),
            pl.BlockSpec((1, E), lambda c: (0, 0)),
            pl.BlockSpec((CI, H), lambda c: (c, 0)),
            pl.BlockSpec((CI, H), lambda c: (c, 0)),
            pl.BlockSpec((EC, H, I), lambda c: (c, 0, 0)),
            pl.BlockSpec((I, H), lambda c: (0, 0)),
            pl.BlockSpec((I, H), lambda c: (0, 0)),
            pl.BlockSpec((H, I), lambda c: (0, 0)),
        ],
        out_specs=[
            pl.BlockSpec((T, H), lambda c: (0, 0)),
            pl.BlockSpec((T, E), lambda c: (0, 0)),
        ],
        out_shape=[
            jax.ShapeDtypeStruct((T, H), jnp.float32),
            jax.ShapeDtypeStruct((T, E), jnp.float32),
        ],
        scratch_shapes=[pltpu.VMEM((T, CI), jnp.float32)],
        compiler_params=pltpu.CompilerParams(
            dimension_semantics=("arbitrary",)),
    )(hidden_states, Wg, bias2, W1r, W3r, W2, Ws1, Ws3, Ws2)
    return out


# speedup vs baseline: 1.1452x; 1.0801x over previous
import jax
import jax.numpy as jnp
from jax import lax
from jax.experimental import pallas as pl
from jax.experimental.pallas import tpu as pltpu

T = 2048
H = 1024
E = 16
I = 256
TOPK = 2
NG = 4
TG = 2
RSF = 2.5

EC = 2
NC = E // EC
CI = EC * I

_NEG = float(jnp.finfo(jnp.float32).min)


def _silu(x):
    return x * jax.nn.sigmoid(x)


def _dotT(a, b):
    return lax.dot_general(a, b, (((1,), (1,)), ((), ())),
                           preferred_element_type=jnp.float32)


def _routing(x, Wg, bias2):
    logits = _dotT(x, Wg)
    scores = jax.nn.sigmoid(logits)
    sfc = scores + bias2
    lane = lax.broadcasted_iota(jnp.int32, (T, E), 1)
    gid = lane // (E // NG)
    gsc = jnp.zeros((T, E), jnp.float32)
    for g in range(NG):
        gs = jnp.where(gid == g, sfc, _NEG)
        m1 = jnp.max(gs, axis=1, keepdims=True)
        m2 = jnp.max(jnp.where(gs >= m1, _NEG, gs), axis=1, keepdims=True)
        gsc = jnp.where(gid == g, m1 + m2, gsc)
    gmax1 = jnp.max(gsc, axis=1, keepdims=True)
    mask1 = gsc >= gmax1
    gmax2 = jnp.max(jnp.where(mask1, _NEG, gsc), axis=1, keepdims=True)
    smask = mask1 | (gsc >= gmax2)
    masked = jnp.where(smask, sfc, _NEG)
    t1 = jnp.max(masked, axis=1, keepdims=True)
    em1 = masked >= t1
    t2 = jnp.max(jnp.where(em1, _NEG, masked), axis=1, keepdims=True)
    sel = em1 | (masked >= t2)
    w = jnp.where(sel, scores, 0.0)
    denom = jnp.sum(w, axis=1, keepdims=True) + 1e-20
    return w / denom * RSF


def _moe_body(x_ref, Wg_ref, b_ref, W1_ref, W3_ref, W2_ref,
              Ws1_ref, Ws3_ref, Ws2_ref, out_ref, comb_ref, hc_ref, xb_ref):
    c = pl.program_id(0)

    @pl.when(c == 0)
    def _():
        x = x_ref[...]
        xb0 = x.astype(jnp.bfloat16)
        xb_ref[...] = xb0
        h = (_silu(_dotT(xb0, Ws1_ref[...].astype(jnp.bfloat16)))
             * _dotT(xb0, Ws3_ref[...].astype(jnp.bfloat16)))
        out_ref[...] = _dotT(h.astype(jnp.bfloat16),
                             Ws2_ref[...].astype(jnp.bfloat16))
        comb_ref[...] = _routing(x, Wg_ref[...], b_ref[...])

    xb = xb_ref[...]
    h1 = _dotT(xb, W1_ref[...].astype(jnp.bfloat16))
    h3 = _dotT(xb, W3_ref[...].astype(jnp.bfloat16))
    comb = comb_ref[...]
    lane = lax.broadcasted_iota(jnp.int32, (T, E), 1)
    for k in range(EC):
        sl = slice(k * I, (k + 1) * I)
        col = jnp.sum(jnp.where(lane == EC * c + k, comb, 0.0),
                      axis=1, keepdims=True)
        hc_ref[:, sl] = (_silu(h1[:, sl]) * h3[:, sl]
                         * col).astype(jnp.bfloat16)
    w2t = jnp.transpose(W2_ref[...], (0, 2, 1)).reshape(CI, H)
    out_ref[...] += jnp.dot(hc_ref[...], w2t.astype(jnp.bfloat16),
                            preferred_element_type=jnp.float32)


def kernel(hidden_states, Wg, bias, W1, W3, W2, Ws1, Ws3, Ws2):
    bias2 = bias.reshape(1, E)
    W1r = W1.reshape(E * I, H)
    W3r = W3.reshape(E * I, H)
    out, _ = pl.pallas_call(
        _moe_body,
        grid=(NC,),
        in_specs=[
            pl.BlockSpec((T, H), lambda c: (0, 0)),
            pl.BlockSpec((E, H), lambda c: (0, 0)),
            pl.BlockSpec((1, E), lambda c: (0, 0)),
            pl.BlockSpec((CI, H), lambda c: (c, 0)),
            pl.BlockSpec((CI, H), lambda c: (c, 0)),
            pl.BlockSpec((EC, H, I), lambda c: (c, 0, 0)),
            pl.BlockSpec((I, H), lambda c: (0, 0)),
            pl.BlockSpec((I, H), lambda c: (0, 0)),
            pl.BlockSpec((H, I), lambda c: (0, 0)),
        ],
        out_specs=[
            pl.BlockSpec((T, H), lambda c: (0, 0)),
            pl.BlockSpec((T, E), lambda c: (0, 0)),
        ],
        out_shape=[
            jax.ShapeDtypeStruct((T, H), jnp.float32),
            jax.ShapeDtypeStruct((T, E), jnp.float32),
        ],
        scratch_shapes=[pltpu.VMEM((T, CI), jnp.bfloat16),
                        pltpu.VMEM((T, H), jnp.bfloat16)],
        compiler_params=pltpu.CompilerParams(
            dimension_semantics=("arbitrary",)),
    )(hidden_states, Wg, bias2, W1r, W3r, W2, Ws1, Ws3, Ws2)
    return out
